# baseline (speedup 1.0000x reference)
import functools

import jax
import jax.numpy as jnp
from jax import lax
from jax.experimental import pallas as pl
from jax.experimental.pallas import tpu as pltpu
from jax.experimental.pallas import tpu_sc as plsc

_B, _C, _H, _W = 4, 96, 384, 384
_P = 16
_HP = _H // _P
_WP = _W // _P
_L = _HP * _WP
_NPATCH = _B * _L
_SEGS = _C * _P
_TROWS = _B * _C * _H * _W // 16
_CSTRIDE = _H * _WP
_BSTRIDE = _C * _CSTRIDE

_NW = 32
_PPW = _NPATCH // _NW
_NBUF = 2
_ROWS128 = _B * _C * _H * _W // 128


def _detile_body(x_ref, o_ref):
    blk = x_ref[...].reshape(_H // 8, 8, _W // 128, 128)
    blk = jnp.transpose(blk, (0, 2, 1, 3))
    o_ref[...] = blk.reshape(_H // 8 * _W // 128 * 8, 128)


def _detile(x):
    n_rows = _H // 8 * _W // 128 * 8
    return pl.pallas_call(
        _detile_body,
        grid=(_B, _C),
        in_specs=[pl.BlockSpec((1, 1, _H, _W), lambda b, c: (b, c, 0, 0))],
        out_specs=pl.BlockSpec((n_rows, 128), lambda b, c: (b * _C + c, 0)),
        out_shape=jax.ShapeDtypeStruct((_ROWS128, 128), jnp.float32),
    )(x)


def _body(x_hbm, perm_hbm, out_hbm, perm_v, base_v, idx_v, data_v, gsems, ssems):
    cid = lax.axis_index("c")
    sid = lax.axis_index("s")
    wid = cid * 16 + sid
    g0 = wid * _PPW
    b = lax.div(g0, _L)

    pltpu.sync_copy(perm_hbm.at[pl.ds(g0, 80)], perm_v)

    for t in range(5):
        sv = perm_v[pl.ds(t * 16, 16)]
        wpv = jnp.full((16,), _WP, jnp.int32)
        e8 = jnp.full((16,), 8, jnp.int32)
        jv = lax.rem(sv, wpv)
        base_v[pl.ds(t * 16, 16)] = (
            jnp.full((16,), b * _BSTRIDE, jnp.int32)
            + lax.div(sv, wpv) * (_P * _WP)
            + lax.div(jv, e8) * 64
            + lax.rem(jv, e8)
        )

    def fill_idx(p, s):
        pq = lax.div(p, 16)
        pr = lax.rem(p, 16)
        chunk = base_v[pl.ds(pq * 16, 16)]
        dnums = lax.GatherDimensionNumbers(
            offset_dims=(), collapsed_slice_dims=(0,), start_index_map=(0,)
        )
        basev = lax.gather(
            chunk,
            jnp.full((16, 1), pr, jnp.int32),
            dnums,
            (1,),
            mode=lax.GatherScatterMode.PROMISE_IN_BOUNDS,
        )

        def fill_c(cq, _):
            lane = lax.iota(jnp.int32, 16)
            e8 = jnp.full((16,), 8, jnp.int32)
            di_off = lax.div(lane, e8) * 192 + lax.rem(lane, e8) * 8
            for u in range(4):
                c = cq * 4 + u
                idx_v[s, pl.ds(c * 16, 16)] = (
                    basev + jnp.full((16,), c * _CSTRIDE, jnp.int32) + di_off
                )
            return 0

        lax.fori_loop(0, _C // 4, fill_c, 0)

    def step_body(st, carry):
        copies = []
        for s in range(_NBUF):
            p = st * _NBUF + s

            @pl.when(st > 0)
            def _drain_scatter(s=s):
                pltpu.make_async_copy(
                    x_hbm.at[pl.ds(0, _SEGS)], data_v.at[s], ssems.at[s]
                ).wait()

            fill_idx(p, s)
            copies.append(
                pltpu.async_copy(
                    x_hbm.at[idx_v.at[s]], data_v.at[s], gsems.at[s]
                )
            )
        for s in range(_NBUF):
            p = st * _NBUF + s
            copies[s].wait()
            pltpu.async_copy(
                data_v.at[s],
                out_hbm.at[pl.ds((g0 + p) * _SEGS, _SEGS)],
                ssems.at[s],
            )
        return carry

    lax.fori_loop(0, _PPW // _NBUF, step_body, 0)

    for s in range(_NBUF):
        pltpu.make_async_copy(
            x_hbm.at[pl.ds(0, _SEGS)], data_v.at[s], ssems.at[s]
        ).wait()


def kernel(x, perm):
    x2d = _detile(x).reshape(_TROWS, 16)
    perm_flat = jnp.pad(perm.reshape(_NPATCH), (0, 16))

    mesh = plsc.VectorSubcoreMesh(core_axis_name="c", subcore_axis_name="s")
    run = functools.partial(
        pl.kernel,
        mesh=mesh,
        out_type=jax.ShapeDtypeStruct((_TROWS, 16), jnp.float32),
        compiler_params=pltpu.CompilerParams(use_tc_tiling_on_sc=False),
        scratch_types=[
            pltpu.VMEM((80,), jnp.int32),
            pltpu.VMEM((80,), jnp.int32),
            pltpu.VMEM((_NBUF, _SEGS), jnp.int32),
            pltpu.VMEM((_NBUF, _SEGS, 16), jnp.float32),
            pltpu.SemaphoreType.DMA((_NBUF,)),
            pltpu.SemaphoreType.DMA((_NBUF,)),
        ],
    )(_body)
    out2d = run(x2d, perm_flat)
    return out2d.reshape(_B, _L, _C, _P, _P)


# speedup vs baseline: 1.0008x; 1.0008x over previous
import functools

import jax
import jax.numpy as jnp
from jax import lax
from jax.experimental import pallas as pl
from jax.experimental.pallas import tpu as pltpu
from jax.experimental.pallas import tpu_sc as plsc

_B, _C, _H, _W = 4, 96, 384, 384
_P = 16
_HP = _H // _P
_WP = _W // _P
_L = _HP * _WP
_NPATCH = _B * _L
_SEGS = _C * _P
_TROWS = _B * _C * _H * _W // 16
_CSTRIDE = _H * _WP
_BSTRIDE = _C * _CSTRIDE

_NW = 32
_PPW = _NPATCH // _NW
_NBUF = 2
_ROWS128 = _B * _C * _H * _W // 128


def _detile_body(x_ref, o_ref):
    blk = x_ref[...].reshape(_H // 8, 8, _W // 128, 128)
    blk = jnp.transpose(blk, (0, 2, 1, 3))
    o_ref[...] = blk.reshape(_H * _W)


def _detile(x):
    n_el = _H * _W
    return pl.pallas_call(
        _detile_body,
        grid=(_B, _C),
        in_specs=[pl.BlockSpec((1, 1, _H, _W), lambda b, c: (b, c, 0, 0))],
        out_specs=pl.BlockSpec((n_el,), lambda b, c: (b * _C + c,)),
        out_shape=jax.ShapeDtypeStruct((_ROWS128 * 128,), jnp.float32),
    )(x)


def _body(x_hbm, perm_hbm, out_hbm, perm_v, base_v, idx_v, data_v, gsems, ssems):
    cid = lax.axis_index("c")
    sid = lax.axis_index("s")
    wid = cid * 16 + sid
    g0 = wid * _PPW
    b = lax.div(g0, _L)

    pltpu.sync_copy(perm_hbm.at[pl.ds(g0, 80)], perm_v)

    for t in range(5):
        sv = perm_v[pl.ds(t * 16, 16)]
        wpv = jnp.full((16,), _WP, jnp.int32)
        e8 = jnp.full((16,), 8, jnp.int32)
        jv = lax.rem(sv, wpv)
        base_v[pl.ds(t * 16, 16)] = (
            jnp.full((16,), b * _BSTRIDE, jnp.int32)
            + lax.div(sv, wpv) * (_P * _WP)
            + lax.div(jv, e8) * 64
            + lax.rem(jv, e8)
        )

    def fill_idx(p, s):
        pq = lax.div(p, 16)
        pr = lax.rem(p, 16)
        chunk = base_v[pl.ds(pq * 16, 16)]
        dnums = lax.GatherDimensionNumbers(
            offset_dims=(), collapsed_slice_dims=(0,), start_index_map=(0,)
        )
        basev = lax.gather(
            chunk,
            jnp.full((16, 1), pr, jnp.int32),
            dnums,
            (1,),
            mode=lax.GatherScatterMode.PROMISE_IN_BOUNDS,
        )

        def fill_c(cq, _):
            lane = lax.iota(jnp.int32, 16)
            e8 = jnp.full((16,), 8, jnp.int32)
            di_off = lax.div(lane, e8) * 192 + lax.rem(lane, e8) * 8
            for u in range(4):
                c = cq * 4 + u
                idx_v[s, pl.ds(c * 16, 16)] = (
                    basev + jnp.full((16,), c * _CSTRIDE, jnp.int32) + di_off
                )
            return 0

        lax.fori_loop(0, _C // 4, fill_c, 0)

    def step_body(st, carry):
        copies = []
        for s in range(_NBUF):
            p = st * _NBUF + s

            @pl.when(st > 0)
            def _drain_scatter(s=s):
                pltpu.make_async_copy(
                    x_hbm.at[pl.ds(0, _SEGS)], data_v.at[s], ssems.at[s]
                ).wait()

            fill_idx(p, s)
            copies.append(
                pltpu.async_copy(
                    x_hbm.at[idx_v.at[s]], data_v.at[s], gsems.at[s]
                )
            )
        for s in range(_NBUF):
            p = st * _NBUF + s
            copies[s].wait()
            pltpu.async_copy(
                data_v.at[s],
                out_hbm.at[pl.ds((g0 + p) * _SEGS, _SEGS)],
                ssems.at[s],
            )
        return carry

    lax.fori_loop(0, _PPW // _NBUF, step_body, 0)

    for s in range(_NBUF):
        pltpu.make_async_copy(
            x_hbm.at[pl.ds(0, _SEGS)], data_v.at[s], ssems.at[s]
        ).wait()


def kernel(x, perm):
    x2d = _detile(x).reshape(_TROWS, 16)
    perm_flat = jnp.pad(perm.reshape(_NPATCH), (0, 16))

    mesh = plsc.VectorSubcoreMesh(core_axis_name="c", subcore_axis_name="s")
    run = functools.partial(
        pl.kernel,
        mesh=mesh,
        out_type=jax.ShapeDtypeStruct((_TROWS, 16), jnp.float32),
        compiler_params=pltpu.CompilerParams(use_tc_tiling_on_sc=False),
        scratch_types=[
            pltpu.VMEM((80,), jnp.int32),
            pltpu.VMEM((80,), jnp.int32),
            pltpu.VMEM((_NBUF, _SEGS), jnp.int32),
            pltpu.VMEM((_NBUF, _SEGS, 16), jnp.float32),
            pltpu.SemaphoreType.DMA((_NBUF,)),
            pltpu.SemaphoreType.DMA((_NBUF,)),
        ],
    )(_body)
    out2d = run(x2d, perm_flat)
    return out2d.reshape(_B, _L, _C, _P, _P)


# speedup vs baseline: 1.2272x; 1.2262x over previous
import functools

import jax
import jax.numpy as jnp
from jax import lax
from jax.experimental import pallas as pl
from jax.experimental.pallas import tpu as pltpu
from jax.experimental.pallas import tpu_sc as plsc

_B, _C, _H, _W = 4, 96, 384, 384
_P = 16
_HP = _H // _P
_WP = _W // _P
_L = _HP * _WP
_BH = 2
_NPATCH = _BH * _L
_SEGS = _C * _P
_TROWS = _BH * _C * _H * _W // 16
_CSTRIDE = _H * _WP
_BSTRIDE = _C * _CSTRIDE

_NW = 32
_PPW = _NPATCH // _NW
_NBUF = 2


def _body(x_hbm, perm_hbm, out_hbm, perm_v, base_v, idx_v, data_v, gsems, ssems):
    cid = lax.axis_index("c")
    sid = lax.axis_index("s")
    wid = cid * 16 + sid
    g0 = wid * _PPW
    b = lax.div(g0, _L)

    d = lax.rem(wid, 2) * 4
    a0 = pl.multiple_of(g0 - d, 8)
    pltpu.sync_copy(perm_hbm.at[pl.ds(a0, 48)], perm_v)

    for t in range(3):
        sv = perm_v[pl.ds(t * 16, 16)]
        wpv = jnp.full((16,), _WP, jnp.int32)
        e8 = jnp.full((16,), 8, jnp.int32)
        jv = lax.rem(sv, wpv)
        base_v[pl.ds(t * 16, 16)] = (
            jnp.full((16,), b * _BSTRIDE, jnp.int32)
            + lax.div(sv, wpv) * (_P * _WP)
            + lax.div(jv, e8) * 64
            + lax.rem(jv, e8)
        )

    def fill_idx(p, s):
        q = p + d
        pq = lax.div(q, 16)
        pr = lax.rem(q, 16)
        chunk = base_v[pl.ds(pq * 16, 16)]
        dnums = lax.GatherDimensionNumbers(
            offset_dims=(), collapsed_slice_dims=(0,), start_index_map=(0,)
        )
        basev = lax.gather(
            chunk,
            jnp.full((16, 1), pr, jnp.int32),
            dnums,
            (1,),
            mode=lax.GatherScatterMode.PROMISE_IN_BOUNDS,
        )

        def fill_c(cq, _):
            lane = lax.iota(jnp.int32, 16)
            e8 = jnp.full((16,), 8, jnp.int32)
            di_off = lax.div(lane, e8) * 192 + lax.rem(lane, e8) * 8
            for u in range(4):
                c = cq * 4 + u
                idx_v[s, pl.ds(c * 16, 16)] = (
                    basev + jnp.full((16,), c * _CSTRIDE, jnp.int32) + di_off
                )
            return 0

        lax.fori_loop(0, _C // 4, fill_c, 0)

    def step_body(st, carry):
        copies = []
        for s in range(_NBUF):
            p = st * _NBUF + s

            @pl.when(st > 0)
            def _drain_scatter(s=s):
                pltpu.make_async_copy(
                    x_hbm.at[pl.ds(0, _SEGS)], data_v.at[s], ssems.at[s]
                ).wait()

            fill_idx(p, s)
            copies.append(
                pltpu.async_copy(
                    x_hbm.at[idx_v.at[s]], data_v.at[s], gsems.at[s]
                )
            )
        for s in range(_NBUF):
            p = st * _NBUF + s
            copies[s].wait()
            pltpu.async_copy(
                data_v.at[s],
                out_hbm.at[pl.ds((g0 + p) * _SEGS, _SEGS)],
                ssems.at[s],
            )
        return carry

    lax.fori_loop(0, _PPW // _NBUF, step_body, 0)

    for s in range(_NBUF):
        pltpu.make_async_copy(
            x_hbm.at[pl.ds(0, _SEGS)], data_v.at[s], ssems.at[s]
        ).wait()


def _make_half():
    mesh = plsc.VectorSubcoreMesh(core_axis_name="c", subcore_axis_name="s")
    return functools.partial(
        pl.kernel,
        mesh=mesh,
        out_type=jax.ShapeDtypeStruct((_TROWS, 16), jnp.float32),
        compiler_params=pltpu.CompilerParams(use_tc_tiling_on_sc=False),
        scratch_types=[
            pltpu.VMEM((48,), jnp.int32),
            pltpu.VMEM((48,), jnp.int32),
            pltpu.VMEM((_NBUF, _SEGS), jnp.int32),
            pltpu.VMEM((_NBUF, _SEGS, 16), jnp.float32),
            pltpu.SemaphoreType.DMA((_NBUF,)),
            pltpu.SemaphoreType.DMA((_NBUF,)),
        ],
    )(_body)


def kernel(x, perm):
    run = _make_half()
    outs = []
    for h in range(_B // _BH):
        xh = x[h * _BH:(h + 1) * _BH]
        x2d = (
            xh.reshape(_BH, _C, _H // 8, 8, _W // 128, 128)
            .transpose(0, 1, 2, 4, 3, 5)
            .reshape(_TROWS, 16)
        )
        perm_h = jnp.pad(perm[h * _BH:(h + 1) * _BH].reshape(_NPATCH), (0, 16))
        outs.append(run(x2d, perm_h).reshape(_BH, _L, _C, _P, _P))
    return jnp.concatenate(outs, axis=0)


# speedup vs baseline: 1.3826x; 1.1266x over previous
import functools

import jax
import jax.numpy as jnp
from jax import lax
from jax.experimental import pallas as pl
from jax.experimental.pallas import tpu as pltpu
from jax.experimental.pallas import tpu_sc as plsc

_B, _C, _H, _W = 4, 96, 384, 384
_P = 16
_HP = _H // _P
_WP = _W // _P
_L = _HP * _WP
_BH = 1
_NPATCH = _BH * _L
_SEGS = _C * _P
_TROWS = _BH * _C * _H * _W // 16
_CSTRIDE = _H * _WP
_BSTRIDE = _C * _CSTRIDE

_NW = 32
_PPW = _NPATCH // _NW
_NBUF = 2


def _body(x_hbm, perm_hbm, out_hbm, perm_v, base_v, idx_v, data_v, gsems, ssems):
    cid = lax.axis_index("c")
    sid = lax.axis_index("s")
    wid = cid * 16 + sid
    g0 = wid * _PPW
    b = lax.div(g0, _L)

    d = lax.rem(g0, 8)
    a0 = pl.multiple_of(g0 - d, 8)
    pltpu.sync_copy(perm_hbm.at[pl.ds(a0, 48)], perm_v)

    for t in range(3):
        sv = perm_v[pl.ds(t * 16, 16)]
        wpv = jnp.full((16,), _WP, jnp.int32)
        e8 = jnp.full((16,), 8, jnp.int32)
        jv = lax.rem(sv, wpv)
        base_v[pl.ds(t * 16, 16)] = (
            jnp.full((16,), b * _BSTRIDE, jnp.int32)
            + lax.div(sv, wpv) * (_P * _WP)
            + lax.div(jv, e8) * 64
            + lax.rem(jv, e8)
        )

    def fill_idx(p, s):
        q = p + d
        pq = lax.div(q, 16)
        pr = lax.rem(q, 16)
        chunk = base_v[pl.ds(pq * 16, 16)]
        dnums = lax.GatherDimensionNumbers(
            offset_dims=(), collapsed_slice_dims=(0,), start_index_map=(0,)
        )
        basev = lax.gather(
            chunk,
            jnp.full((16, 1), pr, jnp.int32),
            dnums,
            (1,),
            mode=lax.GatherScatterMode.PROMISE_IN_BOUNDS,
        )

        def fill_c(cq, _):
            lane = lax.iota(jnp.int32, 16)
            e8 = jnp.full((16,), 8, jnp.int32)
            di_off = lax.div(lane, e8) * 192 + lax.rem(lane, e8) * 8
            for u in range(4):
                c = cq * 4 + u
                idx_v[s, pl.ds(c * 16, 16)] = (
                    basev + jnp.full((16,), c * _CSTRIDE, jnp.int32) + di_off
                )
            return 0

        lax.fori_loop(0, _C // 4, fill_c, 0)

    def step_body(st, carry):
        copies = []
        for s in range(_NBUF):
            p = st * _NBUF + s

            @pl.when(st > 0)
            def _drain_scatter(s=s):
                pltpu.make_async_copy(
                    x_hbm.at[pl.ds(0, _SEGS)], data_v.at[s], ssems.at[s]
                ).wait()

            fill_idx(p, s)
            copies.append(
                pltpu.async_copy(
                    x_hbm.at[idx_v.at[s]], data_v.at[s], gsems.at[s]
                )
            )
        for s in range(_NBUF):
            p = st * _NBUF + s
            copies[s].wait()
            pltpu.async_copy(
                data_v.at[s],
                out_hbm.at[pl.ds((g0 + p) * _SEGS, _SEGS)],
                ssems.at[s],
            )
        return carry

    lax.fori_loop(0, _PPW // _NBUF, step_body, 0)

    for s in range(_NBUF):
        pltpu.make_async_copy(
            x_hbm.at[pl.ds(0, _SEGS)], data_v.at[s], ssems.at[s]
        ).wait()


def _make_half():
    mesh = plsc.VectorSubcoreMesh(core_axis_name="c", subcore_axis_name="s")
    return functools.partial(
        pl.kernel,
        mesh=mesh,
        out_type=jax.ShapeDtypeStruct((_TROWS, 16), jnp.float32),
        compiler_params=pltpu.CompilerParams(use_tc_tiling_on_sc=False),
        scratch_types=[
            pltpu.VMEM((48,), jnp.int32),
            pltpu.VMEM((48,), jnp.int32),
            pltpu.VMEM((_NBUF, _SEGS), jnp.int32),
            pltpu.VMEM((_NBUF, _SEGS, 16), jnp.float32),
            pltpu.SemaphoreType.DMA((_NBUF,)),
            pltpu.SemaphoreType.DMA((_NBUF,)),
        ],
    )(_body)


def kernel(x, perm):
    run = _make_half()
    outs = []
    for h in range(_B // _BH):
        xh = x[h * _BH:(h + 1) * _BH]
        x2d = (
            xh.reshape(_BH, _C, _H // 8, 8, _W // 128, 128)
            .transpose(0, 1, 2, 4, 3, 5)
            .reshape(_TROWS, 16)
        )
        perm_h = jnp.pad(perm[h * _BH:(h + 1) * _BH].reshape(_NPATCH), (0, 48))
        outs.append(run(x2d, perm_h).reshape(_BH, _L, _C, _P, _P))
    return jnp.concatenate(outs, axis=0)
